# Initial kernel scaffold; baseline (speedup 1.0000x reference)
import jax
import jax.numpy as jnp
from jax.experimental import pallas as pl


def kernel(indices, trainable_embedding, fixed_embedding):
    raise NotImplementedError("write your pallas kernel here")


# speedup vs baseline: 3.2285x; 3.2285x over previous
import functools

import jax
import jax.numpy as jnp
from jax import lax
from jax.experimental import pallas as pl
from jax.experimental.pallas import tpu as pltpu
from jax.experimental.pallas import tpu_sc as plsc

NC = 2
NS = 16
NW = NC * NS
L = 16

D = 128
K = 128


def _sc_lookup(idx, trainable, fixed):
    b_total = idx.shape[0]
    train_n = trainable.shape[0]
    rows_per_w = b_total // NW
    n_chunks = rows_per_w // K
    mesh = plsc.VectorSubcoreMesh(core_axis_name="c", subcore_axis_name="s")

    @functools.partial(
        pl.kernel,
        out_type=jax.ShapeDtypeStruct((b_total, D), jnp.float32),
        mesh=mesh,
        scratch_types=[
            pltpu.VMEM((K,), jnp.int32),
            pltpu.VMEM((K,), jnp.int32),
            pltpu.VMEM((K, D), jnp.float32),
            pltpu.VMEM((K + L,), jnp.int32),
            pltpu.VMEM((K + L,), jnp.int32),
            pltpu.SemaphoreType.DMA,
        ],
        compiler_params=pltpu.CompilerParams(needs_layout_passes=False),
    )
    def k(idx_hbm, train_hbm, fixed_hbm, out_hbm, idxv, fidxv, buf, jlist,
          tlist, sem):
        wid = lax.axis_index("s") * NC + lax.axis_index("c")

        def chunk_body(c, _):
            base = wid * rows_per_w + c * K
            pltpu.sync_copy(idx_hbm.at[pl.ds(base, K)], idxv)

            def grp(g, off):
                v = idxv[pl.ds(g * L, L)]
                is_tr = v < train_n
                fidxv[pl.ds(g * L, L)] = jnp.maximum(v - train_n, 0)
                jvec = lax.iota(jnp.int32, L) + g * L
                pfx = plsc.cumsum(is_tr.astype(jnp.int32))
                slots = off + pfx - 1
                plsc.store_scatter(jlist, [slots], jvec, mask=is_tr)
                plsc.store_scatter(tlist, [slots], v, mask=is_tr)
                return off + pfx[L - 1]

            n_tr = lax.fori_loop(0, K // L, grp, jnp.int32(0))
            pltpu.async_copy(fixed_hbm.at[fidxv], buf, sem).wait()

            def patch(i, _):
                j = jlist[pl.ds(i, L)][0]
                t = tlist[pl.ds(i, L)][0]
                pltpu.sync_copy(train_hbm.at[t], buf.at[j])
                return 0

            lax.fori_loop(0, n_tr, patch, 0)
            pltpu.sync_copy(buf, out_hbm.at[pl.ds(base, K)])
            return 0

        lax.fori_loop(0, n_chunks, chunk_body, 0)

    return k(idx, trainable, fixed)


def kernel(indices, trainable_embedding, fixed_embedding):
    b, t = indices.shape
    idx = indices.reshape(-1).astype(jnp.int32)
    out = _sc_lookup(idx, trainable_embedding, fixed_embedding)
    return out.reshape(b, t, D)
